# baseline (speedup 1.0000x reference)
import functools

import jax
import jax.numpy as jnp
from jax.experimental import pallas as pl
from jax.experimental.pallas import tpu as pltpu

_K_BLK = 2048
_NBUF = 4


def _body(xt_hbm, wt_hbm, o_ref, wt_buf, tail_buf, *scratch, k_total, m, n):
    bufs = scratch[:_NBUF]
    sems = scratch[_NBUF]
    aux_sems = scratch[_NBUF + 1]
    nfull = k_total // _K_BLK
    tail = k_total - nfull * _K_BLK
    rounds = nfull // _NBUF
    leftover = nfull - rounds * _NBUF

    def copy(i, s):
        return pltpu.make_async_copy(
            xt_hbm.at[pl.ds(i * _K_BLK, _K_BLK), :], bufs[s], sems.at[s]
        )

    for s in range(min(_NBUF, nfull)):
        copy(s, s).start()
    if tail:
        tail_copy = pltpu.make_async_copy(
            xt_hbm.at[pl.ds(nfull * _K_BLK, tail), :], tail_buf, aux_sems.at[0]
        )
        tail_copy.start()
    w_copy = pltpu.make_async_copy(wt_hbm, wt_buf, aux_sems.at[1])
    w_copy.start()

    o_ref[...] = jnp.zeros_like(o_ref)
    w_copy.wait()

    def contract(xb, wb):
        return jax.lax.dot_general(
            xb, wb, (((0,), (1,)), ((), ())),
            preferred_element_type=jnp.float32,
        )

    def process(i, s):
        copy(i, s).wait()
        wb = wt_buf[:, pl.ds(i * _K_BLK, _K_BLK)]
        o_ref[...] += contract(bufs[s][...], wb)

    def round_body(r, carry):
        for s in range(_NBUF):
            i = r * _NBUF + s
            process(i, s)
            nxt = i + _NBUF

            @pl.when(nxt < nfull)
            def _refill():
                copy(nxt, s).start()
        return carry

    jax.lax.fori_loop(0, rounds, round_body, 0, unroll=False)
    for s in range(leftover):
        process(rounds * _NBUF + s, s)

    if tail:
        tail_copy.wait()
        wb = wt_buf[:, pl.ds(nfull * _K_BLK, tail)]
        o_ref[...] += contract(tail_buf[...], wb)


def kernel(x, W):
    m, k_total = x.shape
    _, n = W.shape
    xt = jnp.transpose(x)
    wt = jnp.transpose(W)
    tail = k_total % _K_BLK
    tail_rows = tail if tail else _K_BLK
    return pl.pallas_call(
        functools.partial(_body, k_total=k_total, m=m, n=n),
        in_specs=[
            pl.BlockSpec(memory_space=pltpu.MemorySpace.HBM),
            pl.BlockSpec(memory_space=pltpu.MemorySpace.HBM),
        ],
        out_specs=pl.BlockSpec(memory_space=pltpu.MemorySpace.VMEM),
        out_shape=jax.ShapeDtypeStruct((m, n), jnp.float32),
        scratch_shapes=[
            pltpu.VMEM((n, k_total), jnp.float32),
            pltpu.VMEM((tail_rows, m), jnp.float32),
        ]
        + [pltpu.VMEM((_K_BLK, m), jnp.float32) for _ in range(_NBUF)]
        + [pltpu.SemaphoreType.DMA((_NBUF,)), pltpu.SemaphoreType.DMA((2,))],
    )(xt, wt)


# speedup vs baseline: 1.0034x; 1.0034x over previous
import functools

import jax
import jax.numpy as jnp
from jax.experimental import pallas as pl
from jax.experimental.pallas import tpu as pltpu

_K_BLK = 6144


def _mm_body(xt_ref, wt_ref, o_ref, *, k_total, nk):
    k = pl.program_id(0)

    @pl.when(k == 0)
    def _init():
        o_ref[...] = jnp.zeros_like(o_ref)

    def contract(xb, wb):
        return jax.lax.dot_general(
            xb, wb, (((0,), (1,)), ((), ())),
            preferred_element_type=jnp.float32,
        )

    @pl.when(k < nk - 1)
    def _full():
        o_ref[...] += contract(xt_ref[...], wt_ref[...])

    @pl.when(k == nk - 1)
    def _tail():
        rem = k_total - (nk - 1) * _K_BLK
        xb = xt_ref[...]
        row = jax.lax.broadcasted_iota(jnp.int32, xb.shape, 0)
        xb = jnp.where(row < rem, xb, 0.0)
        wb = wt_ref[...]
        col = jax.lax.broadcasted_iota(jnp.int32, wb.shape, 1)
        wb = jnp.where(col < rem, wb, 0.0)
        o_ref[...] += contract(xb, wb)


def kernel(x, W):
    m, k_total = x.shape
    _, n = W.shape
    nk = pl.cdiv(k_total, _K_BLK)
    xt = jnp.transpose(x)
    wt = jnp.transpose(W)
    return pl.pallas_call(
        functools.partial(_mm_body, k_total=k_total, nk=nk),
        grid=(nk,),
        in_specs=[
            pl.BlockSpec((_K_BLK, m), lambda k: (k, 0)),
            pl.BlockSpec((n, _K_BLK), lambda k: (0, k)),
        ],
        out_specs=pl.BlockSpec((m, n), lambda k: (0, 0)),
        out_shape=jax.ShapeDtypeStruct((m, n), jnp.float32),
        compiler_params=pltpu.CompilerParams(
            dimension_semantics=("arbitrary",),
        ),
    )(xt, wt)


# speedup vs baseline: 1.0474x; 1.0438x over previous
import functools

import jax
import jax.numpy as jnp
from jax.experimental import pallas as pl
from jax.experimental.pallas import tpu as pltpu

_K_BLK = 3072


def _mm_body(xt_ref, wt_ref, o_ref, *, k_total, nk):
    k = pl.program_id(0)

    @pl.when(k == 0)
    def _init():
        o_ref[...] = jnp.zeros_like(o_ref)

    def contract(xb, wb):
        return jax.lax.dot_general(
            xb, wb, (((0,), (1,)), ((), ())),
            preferred_element_type=jnp.float32,
        )

    @pl.when(k < nk - 1)
    def _full():
        o_ref[...] += contract(xt_ref[...], wt_ref[...])

    @pl.when(k == nk - 1)
    def _tail():
        rem = k_total - (nk - 1) * _K_BLK
        xb = xt_ref[...]
        row = jax.lax.broadcasted_iota(jnp.int32, xb.shape, 0)
        xb = jnp.where(row < rem, xb, 0.0)
        wb = wt_ref[...]
        col = jax.lax.broadcasted_iota(jnp.int32, wb.shape, 1)
        wb = jnp.where(col < rem, wb, 0.0)
        o_ref[...] += contract(xb, wb)


def kernel(x, W):
    m, k_total = x.shape
    _, n = W.shape
    nk = pl.cdiv(k_total, _K_BLK)
    xt = jnp.transpose(x)
    wt = jnp.transpose(W)
    return pl.pallas_call(
        functools.partial(_mm_body, k_total=k_total, nk=nk),
        grid=(nk,),
        in_specs=[
            pl.BlockSpec((_K_BLK, m), lambda k: (k, 0)),
            pl.BlockSpec((n, _K_BLK), lambda k: (0, k)),
        ],
        out_specs=pl.BlockSpec((m, n), lambda k: (0, 0)),
        out_shape=jax.ShapeDtypeStruct((m, n), jnp.float32),
        compiler_params=pltpu.CompilerParams(
            dimension_semantics=("arbitrary",),
        ),
    )(xt, wt)
